# Initial kernel scaffold; baseline (speedup 1.0000x reference)
import jax
import jax.numpy as jnp
from jax.experimental import pallas as pl


def kernel(adj, features, W1, b1, W2, b2, W3, b3, Wm, Wfc1, bfc1, Wfc2, bfc2, Ws, bs):
    raise NotImplementedError("write your pallas kernel here")


# speedup vs baseline: 13.4231x; 13.4231x over previous
import functools

import jax
import jax.numpy as jnp
from jax import lax
from jax.experimental import pallas as pl
from jax.experimental.pallas import tpu as pltpu
from jax.experimental.pallas import tpu_sc as plsc

N = 10000
E = 320000
NC = 2
NS = 16
NW = NC * NS
EPW = E // NW
K = 80
NCHUNK = EPW // K
NP = 10240
RPS = NP // NS


def _sc_mesh():
    return plsc.VectorSubcoreMesh(core_axis_name="c", subcore_axis_name="s")


_SC_PARAMS = pltpu.CompilerParams(use_tc_tiling_on_sc=False)


@functools.partial(
    pl.kernel,
    out_type=jax.ShapeDtypeStruct((NC * NP, 16), jnp.float32),
    mesh=_sc_mesh(),
    scratch_types=[
        pltpu.VMEM((K,), jnp.int32),
        pltpu.VMEM((K, 16), jnp.float32),
        pltpu.VMEM_SHARED((NP, 16), jnp.float32),
    ],
    compiler_params=_SC_PARAMS,
)
def _deg_kernel(dst_hbm, ones_hbm, zeros_hbm, out_hbm, didx, ones_v, acc):
    c = lax.axis_index("c")
    s = lax.axis_index("s")
    w = c * NS + s
    pltpu.sync_copy(ones_hbm, ones_v)
    pltpu.sync_copy(zeros_hbm.at[pl.ds(s * RPS, RPS)],
                    acc.at[pl.ds(s * RPS, RPS)])
    plsc.subcore_barrier()

    def chunk(j, carry):
        base = pl.multiple_of(w * EPW + j * K, 8)
        pltpu.sync_copy(dst_hbm.at[pl.ds(base, K)], didx)
        pltpu.sync_copy(ones_v, acc.at[didx], add=True)
        return carry

    lax.fori_loop(0, NCHUNK, chunk, 0)
    plsc.subcore_barrier()
    pltpu.sync_copy(acc.at[pl.ds(s * RPS, RPS)],
                    out_hbm.at[pl.ds(c * NP + s * RPS, RPS)])


def _make_edge_sum(F):
    @functools.partial(
        pl.kernel,
        out_type=jax.ShapeDtypeStruct((NC * NP, F), jnp.float32),
        mesh=_sc_mesh(),
        scratch_types=[
            pltpu.VMEM((K,), jnp.int32),
            pltpu.VMEM((K,), jnp.int32),
            pltpu.VMEM((K, F), jnp.float32),
            pltpu.VMEM_SHARED((NP, F), jnp.float32),
            pltpu.SemaphoreType.DMA,
        ],
        compiler_params=_SC_PARAMS,
    )
    def edge_sum(src_hbm, dst_hbm, y_hbm, zeros_hbm, out_hbm,
                 sidx, didx, rows, acc, sem):
        c = lax.axis_index("c")
        s = lax.axis_index("s")
        w = c * NS + s
        pltpu.sync_copy(zeros_hbm.at[pl.ds(s * RPS, RPS)],
                        acc.at[pl.ds(s * RPS, RPS)])
        plsc.subcore_barrier()

        def chunk(j, carry):
            base = pl.multiple_of(w * EPW + j * K, 8)
            pltpu.sync_copy(src_hbm.at[pl.ds(base, K)], sidx)
            pltpu.sync_copy(dst_hbm.at[pl.ds(base, K)], didx)
            pltpu.async_copy(y_hbm.at[sidx], rows, sem).wait()
            pltpu.sync_copy(rows, acc.at[didx], add=True)
            return carry

        lax.fori_loop(0, NCHUNK, chunk, 0)
        plsc.subcore_barrier()
        pltpu.sync_copy(acc.at[pl.ds(s * RPS, RPS)],
                        out_hbm.at[pl.ds(c * NP + s * RPS, RPS)])

    return edge_sum


_edge_sum64 = _make_edge_sum(64)
_edge_sum32 = _make_edge_sum(32)


def _prep1_body(degp_ref, x_ref, w1_ref, y1_ref, dinv_ref):
    deg = 1.0 + degp_ref[0, 0:N, 0:1] + degp_ref[1, 0:N, 0:1]
    dinv = lax.rsqrt(deg)
    xw = jnp.dot(x_ref[...], w1_ref[...], preferred_element_type=jnp.float32)
    y1_ref[...] = xw * dinv
    dinv_ref[...] = dinv


def _prep1(degp, x, w1):
    return pl.pallas_call(
        _prep1_body,
        out_shape=(
            jax.ShapeDtypeStruct((N, 64), jnp.float32),
            jax.ShapeDtypeStruct((N, 1), jnp.float32),
        ),
    )(degp, x, w1)


def _comb_body(p_ref, y_ref, dinv_ref, b_ref, wn_ref, out_ref):
    dinv = dinv_ref[...]
    h = dinv * (p_ref[0, 0:N] + p_ref[1, 0:N] + y_ref[...]) + b_ref[...]
    h = jnp.maximum(h, 0.0)
    out_ref[...] = dinv * jnp.dot(h, wn_ref[...],
                                  preferred_element_type=jnp.float32)


def _comb(p, y, dinv, b, wn):
    f2 = wn.shape[1]
    return pl.pallas_call(
        _comb_body,
        out_shape=jax.ShapeDtypeStruct((N, f2), jnp.float32),
    )(p, y, dinv, b, wn)


def _final_body(p_ref, y_ref, dinv_ref, b3_ref, wm_ref, wfc1_ref, bfc1_ref,
                wfc2_ref, bfc2_ref, ws_ref, bs_ref, out_ref):
    dinv = dinv_ref[...]
    h = dinv * (p_ref[0, 0:N] + p_ref[1, 0:N] + y_ref[...]) + b3_ref[...]
    h = jnp.maximum(h, 0.0)
    hmean = jnp.sum(h, axis=0, keepdims=True) * (1.0 / N)
    gc = jnp.dot(hmean, wm_ref[...], preferred_element_type=jnp.float32)
    tg = jnp.tanh(gc)
    logit = jnp.dot(h, tg.T, preferred_element_type=jnp.float32)
    scores = 1.0 / (1.0 + jnp.exp(-logit))
    pooled = jnp.sum(h * scores, axis=0, keepdims=True)
    s1 = jnp.maximum(
        jnp.dot(pooled, wfc1_ref[...], preferred_element_type=jnp.float32)
        + bfc1_ref[...], 0.0)
    s2 = jnp.maximum(
        jnp.dot(s1, wfc2_ref[...], preferred_element_type=jnp.float32)
        + bfc2_ref[...], 0.0)
    logits = jnp.dot(s2, ws_ref[...],
                     preferred_element_type=jnp.float32) + bs_ref[...]
    zmax = jnp.max(logits, axis=1, keepdims=True)
    z = logits - zmax
    out_ref[...] = z - jnp.log(jnp.sum(jnp.exp(z), axis=1, keepdims=True))


def _final(p, y, dinv, b3, wm, wfc1, bfc1, wfc2, bfc2, ws, bs):
    return pl.pallas_call(
        _final_body,
        out_shape=jax.ShapeDtypeStruct((1, bs.shape[1]), jnp.float32),
    )(p, y, dinv, b3, wm, wfc1, bfc1, wfc2, bfc2, ws, bs)


def kernel(adj, features, W1, b1, W2, b2, W3, b3, Wm, Wfc1, bfc1, Wfc2, bfc2,
           Ws, bs):
    src = adj[0].astype(jnp.int32)
    dst = adj[1].astype(jnp.int32)
    z64 = jnp.zeros((NP, 64), jnp.float32)
    z32 = jnp.zeros((NP, 32), jnp.float32)
    z16 = jnp.zeros((NP, 16), jnp.float32)
    ones = jnp.ones((K, 16), jnp.float32)

    degp = _deg_kernel(dst, ones, z16).reshape(2, NP, 16)
    y1, dinv = _prep1(degp, features, W1)
    p1 = _edge_sum64(src, dst, y1, z64).reshape(2, NP, 64)
    y2 = _comb(p1, y1, dinv, b1.reshape(1, -1), W2)
    p2 = _edge_sum32(src, dst, y2, z32).reshape(2, NP, 32)
    y3 = _comb(p2, y2, dinv, b2.reshape(1, -1), W3)
    p3 = _edge_sum32(src, dst, y3, z32).reshape(2, NP, 32)
    return _final(p3, y3, dinv, b3.reshape(1, -1), Wm, Wfc1,
                  bfc1.reshape(1, -1), Wfc2, bfc2.reshape(1, -1), Ws,
                  bs.reshape(1, -1))


# speedup vs baseline: 47.3021x; 3.5239x over previous
import functools

import jax
import jax.numpy as jnp
from jax import lax
from jax.experimental import pallas as pl
from jax.experimental.pallas import tpu as pltpu
from jax.experimental.pallas import tpu_sc as plsc

N = 10000
E = 320000
NC = 2
NS = 16
NW = NC * NS
EPW = E // NW
K = 80
NCHUNK = EPW // K
NB = 5
NP = 10240
RPS = NP // NS


def _sc_mesh():
    return plsc.VectorSubcoreMesh(core_axis_name="c", subcore_axis_name="s")


_SC_PARAMS = pltpu.CompilerParams(use_tc_tiling_on_sc=False)


@functools.partial(
    pl.kernel,
    out_type=jax.ShapeDtypeStruct((NC * NP, 16), jnp.float32),
    mesh=_sc_mesh(),
    scratch_types=[
        pltpu.VMEM((NCHUNK, K), jnp.int32),
        pltpu.VMEM((K, 16), jnp.float32),
        pltpu.VMEM_SHARED((NP, 16), jnp.float32),
        pltpu.SemaphoreType.DMA,
        pltpu.SemaphoreType.DMA,
    ],
    compiler_params=_SC_PARAMS,
)
def _deg_kernel(dst_hbm, ones_hbm, zeros_hbm, out_hbm, didx_v, ones_v, acc,
                isem, ssem):
    c = lax.axis_index("c")
    s = lax.axis_index("s")
    w = c * NS + s
    dd = pltpu.async_copy(dst_hbm.at[w], didx_v, isem)
    pltpu.sync_copy(ones_hbm, ones_v)
    pltpu.sync_copy(zeros_hbm.at[pl.ds(s * RPS, RPS)],
                    acc.at[pl.ds(s * RPS, RPS)])
    dd.wait()
    plsc.subcore_barrier()

    def chunk(j, carry):
        pltpu.async_copy(ones_v, acc.at[didx_v.at[j]], ssem, add=True)
        return carry

    lax.fori_loop(0, NCHUNK, chunk, 0)

    def drain(j, carry):
        pltpu.make_async_copy(ones_v, acc.at[didx_v.at[0]], ssem).wait()
        return carry

    lax.fori_loop(0, NCHUNK, drain, 0)
    plsc.subcore_barrier()
    pltpu.sync_copy(acc.at[pl.ds(s * RPS, RPS)],
                    out_hbm.at[pl.ds(c * NP + s * RPS, RPS)])


def _make_edge_sum(F):
    @functools.partial(
        pl.kernel,
        out_type=jax.ShapeDtypeStruct((NC * NP, F), jnp.float32),
        mesh=_sc_mesh(),
        scratch_types=(
            [pltpu.VMEM((NCHUNK, K), jnp.int32),
             pltpu.VMEM((NCHUNK, K), jnp.int32)]
            + [pltpu.VMEM((K, F), jnp.float32) for _ in range(NB)]
            + [pltpu.VMEM_SHARED((NP, F), jnp.float32)]
            + [pltpu.SemaphoreType.DMA for _ in range(NB + 1)]
        ),
        compiler_params=_SC_PARAMS,
    )
    def edge_sum(src_hbm, dst_hbm, y_hbm, zeros_hbm, out_hbm,
                 sidx_v, didx_v, r0, r1, r2, r3, r4, acc,
                 g0, g1, g2, g3, g4, isem):
        rows = (r0, r1, r2, r3, r4)
        gsem = (g0, g1, g2, g3, g4)
        c = lax.axis_index("c")
        s = lax.axis_index("s")
        w = c * NS + s
        ds = pltpu.async_copy(src_hbm.at[w], sidx_v, isem)
        dd = pltpu.async_copy(dst_hbm.at[w], didx_v, isem)
        pltpu.sync_copy(zeros_hbm.at[pl.ds(s * RPS, RPS)],
                        acc.at[pl.ds(s * RPS, RPS)])
        ds.wait()
        dd.wait()
        for b in range(NB):
            pltpu.async_copy(y_hbm.at[sidx_v.at[b]], rows[b], gsem[b])
        plsc.subcore_barrier()

        def group(g, carry):
            for b in range(NB):
                j = g * NB + b
                pltpu.make_async_copy(y_hbm.at[sidx_v.at[j]], rows[b],
                                      gsem[b]).wait()
                pltpu.sync_copy(rows[b], acc.at[didx_v.at[j]], add=True)

                @pl.when(j + NB < NCHUNK)
                def _():
                    pltpu.async_copy(y_hbm.at[sidx_v.at[j + NB]], rows[b],
                                     gsem[b])
            return carry

        lax.fori_loop(0, NCHUNK // NB, group, 0)
        plsc.subcore_barrier()
        pltpu.sync_copy(acc.at[pl.ds(s * RPS, RPS)],
                        out_hbm.at[pl.ds(c * NP + s * RPS, RPS)])

    return edge_sum


_edge_sum64 = _make_edge_sum(64)
_edge_sum32 = _make_edge_sum(32)


def _prep1_body(degp_ref, x_ref, w1_ref, y1_ref, dinv_ref):
    deg = 1.0 + degp_ref[0, 0:N, 0:1] + degp_ref[1, 0:N, 0:1]
    dinv = lax.rsqrt(deg)
    xw = jnp.dot(x_ref[...], w1_ref[...], preferred_element_type=jnp.float32)
    y1_ref[...] = xw * dinv
    dinv_ref[...] = dinv


def _prep1(degp, x, w1):
    return pl.pallas_call(
        _prep1_body,
        out_shape=(
            jax.ShapeDtypeStruct((N, 64), jnp.float32),
            jax.ShapeDtypeStruct((N, 1), jnp.float32),
        ),
    )(degp, x, w1)


def _comb_body(p_ref, y_ref, dinv_ref, b_ref, wn_ref, out_ref):
    dinv = dinv_ref[...]
    h = dinv * (p_ref[0, 0:N] + p_ref[1, 0:N] + y_ref[...]) + b_ref[...]
    h = jnp.maximum(h, 0.0)
    out_ref[...] = dinv * jnp.dot(h, wn_ref[...],
                                  preferred_element_type=jnp.float32)


def _comb(p, y, dinv, b, wn):
    f2 = wn.shape[1]
    return pl.pallas_call(
        _comb_body,
        out_shape=jax.ShapeDtypeStruct((N, f2), jnp.float32),
    )(p, y, dinv, b, wn)


def _final_body(p_ref, y_ref, dinv_ref, b3_ref, wm_ref, wfc1_ref, bfc1_ref,
                wfc2_ref, bfc2_ref, ws_ref, bs_ref, out_ref):
    dinv = dinv_ref[...]
    h = dinv * (p_ref[0, 0:N] + p_ref[1, 0:N] + y_ref[...]) + b3_ref[...]
    h = jnp.maximum(h, 0.0)
    hmean = jnp.sum(h, axis=0, keepdims=True) * (1.0 / N)
    gc = jnp.dot(hmean, wm_ref[...], preferred_element_type=jnp.float32)
    tg = jnp.tanh(gc)
    logit = jnp.dot(h, tg.T, preferred_element_type=jnp.float32)
    scores = 1.0 / (1.0 + jnp.exp(-logit))
    pooled = jnp.sum(h * scores, axis=0, keepdims=True)
    s1 = jnp.maximum(
        jnp.dot(pooled, wfc1_ref[...], preferred_element_type=jnp.float32)
        + bfc1_ref[...], 0.0)
    s2 = jnp.maximum(
        jnp.dot(s1, wfc2_ref[...], preferred_element_type=jnp.float32)
        + bfc2_ref[...], 0.0)
    logits = jnp.dot(s2, ws_ref[...],
                     preferred_element_type=jnp.float32) + bs_ref[...]
    zmax = jnp.max(logits, axis=1, keepdims=True)
    z = logits - zmax
    out_ref[...] = z - jnp.log(jnp.sum(jnp.exp(z), axis=1, keepdims=True))


def _final(p, y, dinv, b3, wm, wfc1, bfc1, wfc2, bfc2, ws, bs):
    return pl.pallas_call(
        _final_body,
        out_shape=jax.ShapeDtypeStruct((1, bs.shape[1]), jnp.float32),
    )(p, y, dinv, b3, wm, wfc1, bfc1, wfc2, bfc2, ws, bs)


def kernel(adj, features, W1, b1, W2, b2, W3, b3, Wm, Wfc1, bfc1, Wfc2, bfc2,
           Ws, bs):
    src = adj[0].astype(jnp.int32).reshape(NW, NCHUNK, K)
    dst = adj[1].astype(jnp.int32).reshape(NW, NCHUNK, K)
    z64 = jnp.zeros((NP, 64), jnp.float32)
    z32 = jnp.zeros((NP, 32), jnp.float32)
    z16 = jnp.zeros((NP, 16), jnp.float32)
    ones = jnp.ones((K, 16), jnp.float32)

    degp = _deg_kernel(dst, ones, z16).reshape(2, NP, 16)
    y1, dinv = _prep1(degp, features, W1)
    p1 = _edge_sum64(src, dst, y1, z64).reshape(2, NP, 64)
    y2 = _comb(p1, y1, dinv, b1.reshape(1, -1), W2)
    p2 = _edge_sum32(src, dst, y2, z32).reshape(2, NP, 32)
    y3 = _comb(p2, y2, dinv, b2.reshape(1, -1), W3)
    p3 = _edge_sum32(src, dst, y3, z32).reshape(2, NP, 32)
    return _final(p3, y3, dinv, b3.reshape(1, -1), Wm, Wfc1,
                  bfc1.reshape(1, -1), Wfc2, bfc2.reshape(1, -1), Ws,
                  bs.reshape(1, -1))


# speedup vs baseline: 49.2137x; 1.0404x over previous
import functools

import jax
import jax.numpy as jnp
from jax import lax
from jax.experimental import pallas as pl
from jax.experimental.pallas import tpu as pltpu
from jax.experimental.pallas import tpu_sc as plsc

N = 10000
E = 320000
NC = 2
NS = 16
NW = NC * NS
EPW = E // NW
K = 80
NCHUNK = EPW // K
NB = 5
NP = 10240
RPS = NP // NS


def _sc_mesh():
    return plsc.VectorSubcoreMesh(core_axis_name="c", subcore_axis_name="s")


_SC_PARAMS = pltpu.CompilerParams(use_tc_tiling_on_sc=False)


@functools.partial(
    pl.kernel,
    out_type=jax.ShapeDtypeStruct((NC, NP, 16), jnp.float32),
    mesh=_sc_mesh(),
    scratch_types=[
        pltpu.VMEM((NCHUNK, K), jnp.int32),
        pltpu.VMEM((K, 16), jnp.float32),
        pltpu.VMEM_SHARED((NP, 16), jnp.float32),
        pltpu.SemaphoreType.DMA,
        pltpu.SemaphoreType.DMA,
    ],
    compiler_params=_SC_PARAMS,
)
def _deg_kernel(adj_hbm, ones_hbm, zeros_hbm, out_hbm, didx_v, ones_v, acc,
                isem, ssem):
    c = lax.axis_index("c")
    s = lax.axis_index("s")
    w = c * NS + s
    dd = pltpu.async_copy(adj_hbm.at[1, w], didx_v, isem)
    pltpu.sync_copy(ones_hbm, ones_v)
    pltpu.sync_copy(zeros_hbm.at[pl.ds(s * RPS, RPS)],
                    acc.at[pl.ds(s * RPS, RPS)])
    dd.wait()
    plsc.subcore_barrier()

    def chunk(j, carry):
        pltpu.async_copy(ones_v, acc.at[didx_v.at[j]], ssem, add=True)
        return carry

    lax.fori_loop(0, NCHUNK, chunk, 0)

    def drain(j, carry):
        pltpu.make_async_copy(ones_v, acc.at[didx_v.at[0]], ssem).wait()
        return carry

    lax.fori_loop(0, NCHUNK, drain, 0)
    plsc.subcore_barrier()
    pltpu.sync_copy(acc.at[pl.ds(s * RPS, RPS)],
                    out_hbm.at[c, pl.ds(s * RPS, RPS)])


def _make_edge_sum(F):
    @functools.partial(
        pl.kernel,
        out_type=jax.ShapeDtypeStruct((NC, NP, F), jnp.float32),
        mesh=_sc_mesh(),
        scratch_types=(
            [pltpu.VMEM((NCHUNK, K), jnp.int32),
             pltpu.VMEM((NCHUNK, K), jnp.int32)]
            + [pltpu.VMEM((K, F), jnp.float32) for _ in range(NB)]
            + [pltpu.VMEM_SHARED((NP, F), jnp.float32)]
            + [pltpu.SemaphoreType.DMA for _ in range(NB + 1)]
        ),
        compiler_params=_SC_PARAMS,
    )
    def edge_sum(adj_hbm, y_hbm, zeros_hbm, out_hbm,
                 sidx_v, didx_v, r0, r1, r2, r3, r4, acc,
                 g0, g1, g2, g3, g4, isem):
        rows = (r0, r1, r2, r3, r4)
        gsem = (g0, g1, g2, g3, g4)
        c = lax.axis_index("c")
        s = lax.axis_index("s")
        w = c * NS + s
        ds = pltpu.async_copy(adj_hbm.at[0, w], sidx_v, isem)
        dd = pltpu.async_copy(adj_hbm.at[1, w], didx_v, isem)
        pltpu.sync_copy(zeros_hbm.at[pl.ds(s * RPS, RPS)],
                        acc.at[pl.ds(s * RPS, RPS)])
        ds.wait()
        dd.wait()
        for b in range(NB):
            pltpu.async_copy(y_hbm.at[sidx_v.at[b]], rows[b], gsem[b])
        plsc.subcore_barrier()

        def group(g, carry):
            for b in range(NB):
                j = g * NB + b
                pltpu.make_async_copy(y_hbm.at[sidx_v.at[j]], rows[b],
                                      gsem[b]).wait()
                pltpu.sync_copy(rows[b], acc.at[didx_v.at[j]], add=True)

                @pl.when(j + NB < NCHUNK)
                def _():
                    pltpu.async_copy(y_hbm.at[sidx_v.at[j + NB]], rows[b],
                                     gsem[b])
            return carry

        lax.fori_loop(0, NCHUNK // NB, group, 0)
        plsc.subcore_barrier()
        pltpu.sync_copy(acc.at[pl.ds(s * RPS, RPS)],
                        out_hbm.at[c, pl.ds(s * RPS, RPS)])

    return edge_sum


_edge_sum64 = _make_edge_sum(64)
_edge_sum32 = _make_edge_sum(32)


BLK = 2000
G = N // BLK


def _mm1_body(x_ref, w1_ref, xw_ref):
    xw_ref[...] = jnp.dot(x_ref[...], w1_ref[...],
                          preferred_element_type=jnp.float32)


def _mm1(x, w1):
    return pl.pallas_call(
        _mm1_body,
        grid=(G,),
        in_specs=[
            pl.BlockSpec((BLK, 128), lambda i: (i, 0)),
            pl.BlockSpec((128, 64), lambda i: (0, 0)),
        ],
        out_specs=pl.BlockSpec((BLK, 64), lambda i: (i, 0)),
        out_shape=jax.ShapeDtypeStruct((N, 64), jnp.float32),
    )(x, w1)


def _scale1_body(d0_ref, d1_ref, xw_ref, y1_ref, dinv_ref):
    deg = 1.0 + d0_ref[0, :, 0:1] + d1_ref[0, :, 0:1]
    dinv = lax.rsqrt(deg)
    y1_ref[...] = xw_ref[...] * dinv
    dinv_ref[...] = dinv


def _scale1(degp, xw):
    return pl.pallas_call(
        _scale1_body,
        grid=(G,),
        in_specs=[
            pl.BlockSpec((1, BLK, 16), lambda i: (0, i, 0)),
            pl.BlockSpec((1, BLK, 16), lambda i: (1, i, 0)),
            pl.BlockSpec((BLK, 64), lambda i: (i, 0)),
        ],
        out_specs=(
            pl.BlockSpec((BLK, 64), lambda i: (i, 0)),
            pl.BlockSpec((BLK, 1), lambda i: (i, 0)),
        ),
        out_shape=(
            jax.ShapeDtypeStruct((N, 64), jnp.float32),
            jax.ShapeDtypeStruct((N, 1), jnp.float32),
        ),
    )(degp, degp, xw)


def _comb_body(p0_ref, p1_ref, y_ref, dinv_ref, b_ref, wn_ref, out_ref):
    dinv = dinv_ref[...]
    h = dinv * (p0_ref[0] + p1_ref[0] + y_ref[...]) + b_ref[...]
    h = jnp.maximum(h, 0.0)
    out_ref[...] = dinv * jnp.dot(h, wn_ref[...],
                                  preferred_element_type=jnp.float32)


def _comb(p, y, dinv, b, wn):
    f = y.shape[1]
    f2 = wn.shape[1]
    return pl.pallas_call(
        _comb_body,
        grid=(G,),
        in_specs=[
            pl.BlockSpec((1, BLK, f), lambda i: (0, i, 0)),
            pl.BlockSpec((1, BLK, f), lambda i: (1, i, 0)),
            pl.BlockSpec((BLK, f), lambda i: (i, 0)),
            pl.BlockSpec((BLK, 1), lambda i: (i, 0)),
            pl.BlockSpec((1, f), lambda i: (0, 0)),
            pl.BlockSpec((f, f2), lambda i: (0, 0)),
        ],
        out_specs=pl.BlockSpec((BLK, f2), lambda i: (i, 0)),
        out_shape=jax.ShapeDtypeStruct((N, f2), jnp.float32),
    )(p, p, y, dinv, b, wn)


def _final_body(p_ref, y_ref, dinv_ref, b3_ref, wm_ref, wfc1_ref, bfc1_ref,
                wfc2_ref, bfc2_ref, ws_ref, bs_ref, out_ref):
    dinv = dinv_ref[...]
    h = dinv * (p_ref[0, 0:N] + p_ref[1, 0:N] + y_ref[...]) + b3_ref[...]
    h = jnp.maximum(h, 0.0)
    hmean = jnp.sum(h, axis=0, keepdims=True) * (1.0 / N)
    gc = jnp.dot(hmean, wm_ref[...], preferred_element_type=jnp.float32)
    tg = jnp.tanh(gc)
    logit = jnp.dot(h, tg.T, preferred_element_type=jnp.float32)
    scores = 1.0 / (1.0 + jnp.exp(-logit))
    pooled = jnp.sum(h * scores, axis=0, keepdims=True)
    s1 = jnp.maximum(
        jnp.dot(pooled, wfc1_ref[...], preferred_element_type=jnp.float32)
        + bfc1_ref[...], 0.0)
    s2 = jnp.maximum(
        jnp.dot(s1, wfc2_ref[...], preferred_element_type=jnp.float32)
        + bfc2_ref[...], 0.0)
    logits = jnp.dot(s2, ws_ref[...],
                     preferred_element_type=jnp.float32) + bs_ref[...]
    zmax = jnp.max(logits, axis=1, keepdims=True)
    z = logits - zmax
    out_ref[...] = z - jnp.log(jnp.sum(jnp.exp(z), axis=1, keepdims=True))


def _final(p, y, dinv, b3, wm, wfc1, bfc1, wfc2, bfc2, ws, bs):
    return pl.pallas_call(
        _final_body,
        out_shape=jax.ShapeDtypeStruct((1, bs.shape[1]), jnp.float32),
    )(p, y, dinv, b3, wm, wfc1, bfc1, wfc2, bfc2, ws, bs)


def kernel(adj, features, W1, b1, W2, b2, W3, b3, Wm, Wfc1, bfc1, Wfc2, bfc2,
           Ws, bs):
    adj4 = adj.astype(jnp.int32).reshape(2, NW, NCHUNK, K)
    z64 = jnp.zeros((NP, 64), jnp.float32)
    z32 = jnp.zeros((NP, 32), jnp.float32)
    z16 = jnp.zeros((NP, 16), jnp.float32)
    ones = jnp.ones((K, 16), jnp.float32)

    degp = _deg_kernel(adj4, ones, z16)
    xw1 = _mm1(features, W1)
    y1, dinv = _scale1(degp, xw1)
    p1 = _edge_sum64(adj4, y1, z64)
    y2 = _comb(p1, y1, dinv, b1.reshape(1, -1), W2)
    p2 = _edge_sum32(adj4, y2, z32)
    y3 = _comb(p2, y2, dinv, b2.reshape(1, -1), W3)
    p3 = _edge_sum32(adj4, y3, z32)
    return _final(p3, y3, dinv, b3.reshape(1, -1), Wm, Wfc1,
                  bfc1.reshape(1, -1), Wfc2, bfc2.reshape(1, -1), Ws,
                  bs.reshape(1, -1))


# speedup vs baseline: 50.8794x; 1.0338x over previous
import functools

import jax
import jax.numpy as jnp
from jax import lax
from jax.experimental import pallas as pl
from jax.experimental.pallas import tpu as pltpu
from jax.experimental.pallas import tpu_sc as plsc

N = 10000
E = 320000
NC = 2
NS = 16
NW = NC * NS
EPW = E // NW
K = 128
NCHUNK = EPW // K
KT = EPW - NCHUNK * K
NB = 6
NP = 10240
RPS = NP // NS


def _sc_mesh():
    return plsc.VectorSubcoreMesh(core_axis_name="c", subcore_axis_name="s")


_SC_PARAMS = pltpu.CompilerParams(use_tc_tiling_on_sc=False)


@functools.partial(
    pl.kernel,
    out_type=jax.ShapeDtypeStruct((NC, NP, 1), jnp.float32),
    mesh=_sc_mesh(),
    scratch_types=[
        pltpu.VMEM((NCHUNK, K), jnp.int32),
        pltpu.VMEM((KT,), jnp.int32),
        pltpu.VMEM((K, 1), jnp.float32),
        pltpu.VMEM((KT, 1), jnp.float32),
        pltpu.VMEM_SHARED((NP, 1), jnp.float32),
        pltpu.SemaphoreType.DMA,
        pltpu.SemaphoreType.DMA,
    ],
    compiler_params=_SC_PARAMS,
)
def _deg_kernel(adjm_hbm, adjt_hbm, ones_hbm, zeros_hbm, out_hbm,
                didx_v, didx_t, ones_v, ones_t, acc, isem, ssem):
    c = lax.axis_index("c")
    s = lax.axis_index("s")
    w = c * NS + s
    dd = pltpu.async_copy(adjm_hbm.at[1, w], didx_v, isem)
    dt = pltpu.async_copy(adjt_hbm.at[1, w], didx_t, isem)
    pltpu.sync_copy(ones_hbm.at[pl.ds(0, K)], ones_v)
    pltpu.sync_copy(ones_hbm.at[pl.ds(0, KT)], ones_t)
    pltpu.sync_copy(zeros_hbm.at[pl.ds(s * RPS, RPS)],
                    acc.at[pl.ds(s * RPS, RPS)])
    dd.wait()
    dt.wait()
    plsc.subcore_barrier()

    def chunk(j, carry):
        pltpu.async_copy(ones_v, acc.at[didx_v.at[j]], ssem, add=True)
        return carry

    lax.fori_loop(0, NCHUNK, chunk, 0)
    pltpu.sync_copy(ones_t, acc.at[didx_t], add=True)

    def drain(j, carry):
        pltpu.make_async_copy(ones_v, acc.at[didx_v.at[0]], ssem).wait()
        return carry

    lax.fori_loop(0, NCHUNK, drain, 0)
    plsc.subcore_barrier()
    pltpu.sync_copy(acc.at[pl.ds(s * RPS, RPS)],
                    out_hbm.at[c, pl.ds(s * RPS, RPS)])


def _make_edge_sum(F):
    @functools.partial(
        pl.kernel,
        out_type=jax.ShapeDtypeStruct((NC, NP, F), jnp.float32),
        mesh=_sc_mesh(),
        scratch_types=(
            [pltpu.VMEM((NCHUNK, K), jnp.int32),
             pltpu.VMEM((NCHUNK, K), jnp.int32),
             pltpu.VMEM((KT,), jnp.int32),
             pltpu.VMEM((KT,), jnp.int32),
             pltpu.VMEM((KT, F), jnp.float32)]
            + [pltpu.VMEM((K, F), jnp.float32) for _ in range(NB)]
            + [pltpu.VMEM_SHARED((NP, F), jnp.float32)]
            + [pltpu.SemaphoreType.DMA for _ in range(NB + 1)]
        ),
        compiler_params=_SC_PARAMS,
    )
    def edge_sum(adjm_hbm, adjt_hbm, y_hbm, zeros_hbm, out_hbm,
                 sidx_v, didx_v, sidx_t, didx_t, rows_t,
                 r0, r1, r2, r3, r4, r5, acc,
                 g0, g1, g2, g3, g4, g5, isem):
        rows = (r0, r1, r2, r3, r4, r5)
        gsem = (g0, g1, g2, g3, g4, g5)
        c = lax.axis_index("c")
        s = lax.axis_index("s")
        w = c * NS + s
        ds = pltpu.async_copy(adjm_hbm.at[0, w], sidx_v, isem)
        dd = pltpu.async_copy(adjm_hbm.at[1, w], didx_v, isem)
        dst_ = pltpu.async_copy(adjt_hbm.at[0, w], sidx_t, isem)
        ddt = pltpu.async_copy(adjt_hbm.at[1, w], didx_t, isem)
        pltpu.sync_copy(zeros_hbm.at[pl.ds(s * RPS, RPS)],
                        acc.at[pl.ds(s * RPS, RPS)])
        ds.wait()
        dd.wait()
        dst_.wait()
        ddt.wait()
        for b in range(NB):
            pltpu.async_copy(y_hbm.at[sidx_v.at[b]], rows[b], gsem[b])
        tl = pltpu.async_copy(y_hbm.at[sidx_t], rows_t, isem)
        plsc.subcore_barrier()

        def group(g, carry):
            for b in range(NB):
                j = g * NB + b
                pltpu.make_async_copy(y_hbm.at[sidx_v.at[j]], rows[b],
                                      gsem[b]).wait()
                pltpu.sync_copy(rows[b], acc.at[didx_v.at[j]], add=True)

                @pl.when(j + NB < NCHUNK)
                def _():
                    pltpu.async_copy(y_hbm.at[sidx_v.at[j + NB]], rows[b],
                                     gsem[b])
            return carry

        lax.fori_loop(0, NCHUNK // NB, group, 0)
        tl.wait()
        pltpu.sync_copy(rows_t, acc.at[didx_t], add=True)
        plsc.subcore_barrier()
        pltpu.sync_copy(acc.at[pl.ds(s * RPS, RPS)],
                        out_hbm.at[c, pl.ds(s * RPS, RPS)])

    return edge_sum


_edge_sum64 = _make_edge_sum(64)
_edge_sum32 = _make_edge_sum(32)


BLK = 2000
G = N // BLK


def _mm1_body(x_ref, w1_ref, xw_ref):
    xw_ref[...] = jnp.dot(x_ref[...], w1_ref[...],
                          preferred_element_type=jnp.float32)


def _mm1(x, w1):
    return pl.pallas_call(
        _mm1_body,
        grid=(G,),
        in_specs=[
            pl.BlockSpec((BLK, 128), lambda i: (i, 0)),
            pl.BlockSpec((128, 64), lambda i: (0, 0)),
        ],
        out_specs=pl.BlockSpec((BLK, 64), lambda i: (i, 0)),
        out_shape=jax.ShapeDtypeStruct((N, 64), jnp.float32),
    )(x, w1)


def _scale1_body(d0_ref, d1_ref, xw_ref, y1_ref, dinv_ref):
    deg = 1.0 + d0_ref[0] + d1_ref[0]
    dinv = lax.rsqrt(deg)
    y1_ref[...] = xw_ref[...] * dinv
    dinv_ref[...] = dinv


def _scale1(degp, xw):
    return pl.pallas_call(
        _scale1_body,
        grid=(G,),
        in_specs=[
            pl.BlockSpec((1, BLK, 1), lambda i: (0, i, 0)),
            pl.BlockSpec((1, BLK, 1), lambda i: (1, i, 0)),
            pl.BlockSpec((BLK, 64), lambda i: (i, 0)),
        ],
        out_specs=(
            pl.BlockSpec((BLK, 64), lambda i: (i, 0)),
            pl.BlockSpec((BLK, 1), lambda i: (i, 0)),
        ),
        out_shape=(
            jax.ShapeDtypeStruct((N, 64), jnp.float32),
            jax.ShapeDtypeStruct((N, 1), jnp.float32),
        ),
    )(degp, degp, xw)


def _comb_body(p0_ref, p1_ref, y_ref, dinv_ref, b_ref, wn_ref, out_ref):
    dinv = dinv_ref[...]
    h = dinv * (p0_ref[0] + p1_ref[0] + y_ref[...]) + b_ref[...]
    h = jnp.maximum(h, 0.0)
    out_ref[...] = dinv * jnp.dot(h, wn_ref[...],
                                  preferred_element_type=jnp.float32)


def _comb(p, y, dinv, b, wn):
    f = y.shape[1]
    f2 = wn.shape[1]
    return pl.pallas_call(
        _comb_body,
        grid=(G,),
        in_specs=[
            pl.BlockSpec((1, BLK, f), lambda i: (0, i, 0)),
            pl.BlockSpec((1, BLK, f), lambda i: (1, i, 0)),
            pl.BlockSpec((BLK, f), lambda i: (i, 0)),
            pl.BlockSpec((BLK, 1), lambda i: (i, 0)),
            pl.BlockSpec((1, f), lambda i: (0, 0)),
            pl.BlockSpec((f, f2), lambda i: (0, 0)),
        ],
        out_specs=pl.BlockSpec((BLK, f2), lambda i: (i, 0)),
        out_shape=jax.ShapeDtypeStruct((N, f2), jnp.float32),
    )(p, p, y, dinv, b, wn)


def _final_body(p_ref, y_ref, dinv_ref, b3_ref, wm_ref, wfc1_ref, bfc1_ref,
                wfc2_ref, bfc2_ref, ws_ref, bs_ref, out_ref):
    dinv = dinv_ref[...]
    h = dinv * (p_ref[0, 0:N] + p_ref[1, 0:N] + y_ref[...]) + b3_ref[...]
    h = jnp.maximum(h, 0.0)
    hmean = jnp.sum(h, axis=0, keepdims=True) * (1.0 / N)
    gc = jnp.dot(hmean, wm_ref[...], preferred_element_type=jnp.float32)
    tg = jnp.tanh(gc)
    logit = jnp.dot(h, tg.T, preferred_element_type=jnp.float32)
    scores = 1.0 / (1.0 + jnp.exp(-logit))
    pooled = jnp.sum(h * scores, axis=0, keepdims=True)
    s1 = jnp.maximum(
        jnp.dot(pooled, wfc1_ref[...], preferred_element_type=jnp.float32)
        + bfc1_ref[...], 0.0)
    s2 = jnp.maximum(
        jnp.dot(s1, wfc2_ref[...], preferred_element_type=jnp.float32)
        + bfc2_ref[...], 0.0)
    logits = jnp.dot(s2, ws_ref[...],
                     preferred_element_type=jnp.float32) + bs_ref[...]
    zmax = jnp.max(logits, axis=1, keepdims=True)
    z = logits - zmax
    out_ref[...] = z - jnp.log(jnp.sum(jnp.exp(z), axis=1, keepdims=True))


def _final(p, y, dinv, b3, wm, wfc1, bfc1, wfc2, bfc2, ws, bs):
    return pl.pallas_call(
        _final_body,
        out_shape=jax.ShapeDtypeStruct((1, bs.shape[1]), jnp.float32),
    )(p, y, dinv, b3, wm, wfc1, bfc1, wfc2, bfc2, ws, bs)


def kernel(adj, features, W1, b1, W2, b2, W3, b3, Wm, Wfc1, bfc1, Wfc2, bfc2,
           Ws, bs):
    adj32 = adj.astype(jnp.int32)
    nm = NW * NCHUNK * K
    adjm = adj32[:, :nm].reshape(2, NW, NCHUNK, K)
    adjt = adj32[:, nm:].reshape(2, NW, KT)
    z64 = jnp.zeros((NP, 64), jnp.float32)
    z32 = jnp.zeros((NP, 32), jnp.float32)
    z1 = jnp.zeros((NP, 1), jnp.float32)
    ones = jnp.ones((K, 1), jnp.float32)

    degp = _deg_kernel(adjm, adjt, ones, z1)
    xw1 = _mm1(features, W1)
    y1, dinv = _scale1(degp, xw1)
    p1 = _edge_sum64(adjm, adjt, y1, z64)
    y2 = _comb(p1, y1, dinv, b1.reshape(1, -1), W2)
    p2 = _edge_sum32(adjm, adjt, y2, z32)
    y3 = _comb(p2, y2, dinv, b2.reshape(1, -1), W3)
    p3 = _edge_sum32(adjm, adjt, y3, z32)
    return _final(p3, y3, dinv, b3.reshape(1, -1), Wm, Wfc1,
                  bfc1.reshape(1, -1), Wfc2, bfc2.reshape(1, -1), Ws,
                  bs.reshape(1, -1))
